# Initial kernel scaffold; baseline (speedup 1.0000x reference)
import functools

import jax
import jax.numpy as jnp
from jax import lax
from jax.experimental import pallas as pl
from jax.experimental.pallas import tpu as pltpu
from jax.experimental.pallas import tpu_sc as plsc

_NC = 2
_NS = 16
_NW = _NC * _NS
_CH = 80


def _sc_body(nch, rows_per_tile, x_hbm, src_hbm, dst_hbm, w_hbm, out_hbm,
             acc, src_v, dst_v, w_v, rows_v, zbuf, sem):
    c = lax.axis_index("c")
    s = lax.axis_index("s")
    tid = c * _NS + s

    zeros16 = jnp.zeros((16,), jnp.float32)
    zrows = zbuf.shape[0]

    def zrow(i, carry):
        for g in range(8):
            zbuf[i, pl.ds(g * 16, 16)] = zeros16
        return carry

    lax.fori_loop(0, zrows, zrow, 0)

    def zcp(i, carry):
        pltpu.sync_copy(zbuf, acc.at[pl.ds(s * rows_per_tile + i * zrows, zrows)])
        return carry

    lax.fori_loop(0, rows_per_tile // zrows, zcp, 0)
    plsc.subcore_barrier()

    pltpu.sync_copy(src_hbm.at[tid], src_v)
    pltpu.sync_copy(dst_hbm.at[tid], dst_v)
    pltpu.sync_copy(w_hbm.at[tid], w_v)

    def chunk(j, carry):
        pltpu.async_copy(x_hbm.at[src_v.at[j]], rows_v, sem).wait()

        def edge(k, carry2):
            wk = plsc.load_gather(
                w_v, [jnp.full((16,), j, jnp.int32), jnp.full((16,), k, jnp.int32)])
            for g in range(8):
                rows_v[k, pl.ds(g * 16, 16)] = rows_v[k, pl.ds(g * 16, 16)] * wk
            return carry2

        lax.fori_loop(0, _CH, edge, 0)
        pltpu.sync_copy(rows_v, acc.at[dst_v.at[j]], add=True)
        return carry

    lax.fori_loop(0, nch, chunk, 0)
    plsc.subcore_barrier()

    pltpu.sync_copy(acc.at[pl.ds(s * rows_per_tile, rows_per_tile)],
                    out_hbm.at[c, pl.ds(s * rows_per_tile, rows_per_tile)])


def _mm_body(p_ref, w_ref, o_ref):
    a = p_ref[0] + p_ref[1]
    o_ref[...] = lax.dot_general(a, w_ref[...], (((1,), (1,)), ((), ())),
                                 preferred_element_type=jnp.float32)


def kernel(node_emb, edge_index, edge_weight, W):
    n, d = node_emb.shape
    e = edge_index.shape[1]
    assert d == 128 and e % (_NW * _CH) == 0 and n % _NS == 0
    nch = e // (_NW * _CH)
    rows_per_tile = n // _NS

    src = edge_index[0].astype(jnp.int32).reshape(_NW, nch, _CH)
    dst = edge_index[1].astype(jnp.int32).reshape(_NW, nch, _CH)
    w3 = edge_weight.reshape(_NW, nch, _CH)

    mesh = plsc.VectorSubcoreMesh(core_axis_name="c", subcore_axis_name="s")
    partials = pl.kernel(
        functools.partial(_sc_body, nch, rows_per_tile),
        out_type=jax.ShapeDtypeStruct((_NC, n, d), jnp.float32),
        mesh=mesh,
        scratch_types=[
            pltpu.VMEM_SHARED((n, d), jnp.float32),
            pltpu.VMEM((nch, _CH), jnp.int32),
            pltpu.VMEM((nch, _CH), jnp.int32),
            pltpu.VMEM((nch, _CH), jnp.float32),
            pltpu.VMEM((_CH, d), jnp.float32),
            pltpu.VMEM((125, d), jnp.float32),
            pltpu.SemaphoreType.DMA,
        ],
    )(node_emb, src, dst, w3)

    bn = 1000
    out = pl.pallas_call(
        _mm_body,
        grid=(n // bn,),
        in_specs=[
            pl.BlockSpec((_NC, bn, d), lambda i: (0, i, 0)),
            pl.BlockSpec((d, d), lambda i: (0, 0)),
        ],
        out_specs=pl.BlockSpec((bn, d), lambda i: (i, 0)),
        out_shape=jax.ShapeDtypeStruct((n, d), jnp.float32),
    )(partials, W)
    return out


# speedup vs baseline: 6.6209x; 6.6209x over previous
import functools

import jax
import jax.numpy as jnp
from jax import lax
from jax.experimental import pallas as pl
from jax.experimental.pallas import tpu as pltpu
from jax.experimental.pallas import tpu_sc as plsc

_NC = 2
_NS = 16
_NW = _NC * _NS
_CH = 80
_SB = 25


def _sc_body(nch, stripe, tail, x_hbm, src_hbm, dst_hbm, w_hbm, out_hbm,
             acc, src_v, dst_v, w_v, rows_v, zbuf, sem):
    c = lax.axis_index("c")
    s = lax.axis_index("s")
    tid = c * _NS + s

    zeros16 = jnp.zeros((16,), jnp.float32)
    zrows = zbuf.shape[0]

    def zrow(i, carry):
        for g in range(8):
            zbuf[i, pl.ds(g * 16, 16)] = zeros16
        return carry

    lax.fori_loop(0, zrows, zrow, 0)

    def zcp(i, carry):
        pltpu.sync_copy(zbuf, acc.at[pl.ds(s * stripe + i * zrows, zrows)])
        return carry

    lax.fori_loop(0, stripe // zrows, zcp, 0)
    if tail:
        @pl.when(s == 0)
        def _():
            pltpu.sync_copy(zbuf.at[pl.ds(0, tail)],
                            acc.at[pl.ds(_NS * stripe, tail)])
    plsc.subcore_barrier()

    def superchunk(sj, carry):
        pltpu.sync_copy(src_hbm.at[tid, sj], src_v)
        pltpu.sync_copy(dst_hbm.at[tid, sj], dst_v)
        pltpu.sync_copy(w_hbm.at[tid, sj], w_v)

        def chunk(j, carry1):
            pltpu.async_copy(x_hbm.at[src_v.at[j]], rows_v, sem).wait()

            def edge16(k16, carry2):
                w16 = w_v[j, pl.ds(k16 * 16, 16)]
                for i in range(16):
                    wk = lax.broadcast_in_dim(
                        lax.squeeze(lax.slice(w16, (i,), (i + 1,)), (0,)), (16,), ())
                    k = k16 * 16 + i
                    for g in range(8):
                        rows_v[k, pl.ds(g * 16, 16)] = (
                            rows_v[k, pl.ds(g * 16, 16)] * wk)
                return carry2

            lax.fori_loop(0, _CH // 16, edge16, 0)
            pltpu.sync_copy(rows_v, acc.at[dst_v.at[j]], add=True)
            return carry1

        lax.fori_loop(0, _SB, chunk, 0)
        return carry

    lax.fori_loop(0, nch // _SB, superchunk, 0)
    plsc.subcore_barrier()

    pltpu.sync_copy(acc.at[pl.ds(s * stripe, stripe)],
                    out_hbm.at[c, pl.ds(s * stripe, stripe)])
    if tail:
        @pl.when(s == 0)
        def _():
            pltpu.sync_copy(acc.at[pl.ds(_NS * stripe, tail)],
                            out_hbm.at[c, pl.ds(_NS * stripe, tail)])


def _mm_body(p_ref, w_ref, o_ref):
    a = p_ref[0] + p_ref[1]
    o_ref[...] = lax.dot_general(a, w_ref[...], (((1,), (1,)), ((), ())),
                                 preferred_element_type=jnp.float32)


def kernel(node_emb, edge_index, edge_weight, W):
    n, d = node_emb.shape
    e = edge_index.shape[1]
    assert d == 128 and e % (_NW * _SB * _CH) == 0
    nch = e // (_NW * _CH)
    nsb = nch // _SB
    stripe = (n // _NS) // 8 * 8
    tail = n - stripe * _NS
    zr = 16
    assert stripe % zr == 0 and tail <= zr

    src = edge_index[0].astype(jnp.int32).reshape(_NW, nsb, _SB, _CH)
    dst = edge_index[1].astype(jnp.int32).reshape(_NW, nsb, _SB, _CH)
    w3 = edge_weight.reshape(_NW, nsb, _SB, _CH)

    mesh = plsc.VectorSubcoreMesh(core_axis_name="c", subcore_axis_name="s")
    partials = pl.kernel(
        functools.partial(_sc_body, nch, stripe, tail),
        out_type=jax.ShapeDtypeStruct((_NC, n, d), jnp.float32),
        mesh=mesh,
        scratch_types=[
            pltpu.VMEM_SHARED((n, d), jnp.float32),
            pltpu.VMEM((_SB, _CH), jnp.int32),
            pltpu.VMEM((_SB, _CH), jnp.int32),
            pltpu.VMEM((_SB, _CH), jnp.float32),
            pltpu.VMEM((_CH, d), jnp.float32),
            pltpu.VMEM((zr, d), jnp.float32),
            pltpu.SemaphoreType.DMA,
        ],
    )(node_emb, src, dst, w3)

    bn = 1000
    out = pl.pallas_call(
        _mm_body,
        grid=(n // bn,),
        in_specs=[
            pl.BlockSpec((_NC, bn, d), lambda i: (0, i, 0)),
            pl.BlockSpec((d, d), lambda i: (0, 0)),
        ],
        out_specs=pl.BlockSpec((bn, d), lambda i: (i, 0)),
        out_shape=jax.ShapeDtypeStruct((n, d), jnp.float32),
    )(partials, W)
    return out
